# baseline (speedup 1.0000x reference)
import functools

import jax
import jax.numpy as jnp
from jax import lax
from jax.experimental import pallas as pl
from jax.experimental.pallas import tpu as pltpu
from jax.experimental.pallas import tpu_sc as plsc

_BATCH = 16384
_D = 32
_CHUNK = 128


@functools.cache
def _gather4():
    info = plsc.get_sparse_core_info()
    nw = info.num_cores * info.num_subcores
    b_per_w = _BATCH // nw
    n_ch = b_per_w // _CHUNK
    mesh = plsc.VectorSubcoreMesh(core_axis_name="c", subcore_axis_name="s")

    @functools.partial(
        pl.kernel,
        out_type=[jax.ShapeDtypeStruct((_BATCH, _D), jnp.float32)] * 4,
        mesh=mesh,
        compiler_params=pltpu.CompilerParams(use_tc_tiling_on_sc=False),
        scratch_types=[
            pltpu.VMEM((n_ch, _CHUNK), jnp.int32),
            pltpu.VMEM((n_ch, _CHUNK), jnp.int32),
            pltpu.VMEM((b_per_w, _D), jnp.float32),
            pltpu.VMEM((b_per_w, _D), jnp.float32),
            pltpu.VMEM((b_per_w, _D), jnp.float32),
            pltpu.VMEM((b_per_w, _D), jnp.float32),
            pltpu.SemaphoreType.DMA,
            pltpu.SemaphoreType.DMA,
        ],
    )
    def gk(uidx_hbm, iidx_hbm, ug_hbm, ig_hbm, um_hbm, im_hbm,
           oug, oig, oum, oim,
           uidx_v, iidx_v, bug, big, bum, bim, gsem, ssem):
        wid = lax.axis_index("s") * info.num_cores + lax.axis_index("c")
        base = wid * b_per_w
        idx_cp = []
        for j in range(n_ch):
            sl = pl.ds(base + j * _CHUNK, _CHUNK)
            idx_cp.append(pltpu.async_copy(uidx_hbm.at[sl], uidx_v.at[j], ssem))
            idx_cp.append(pltpu.async_copy(iidx_hbm.at[sl], iidx_v.at[j], ssem))
        for c in idx_cp:
            c.wait()
        gathers = []
        for j in range(n_ch):
            dst = pl.ds(j * _CHUNK, _CHUNK)
            gathers.append(pltpu.async_copy(ug_hbm.at[uidx_v.at[j]], bug.at[dst], gsem))
            gathers.append(pltpu.async_copy(ig_hbm.at[iidx_v.at[j]], big.at[dst], gsem))
            gathers.append(pltpu.async_copy(um_hbm.at[uidx_v.at[j]], bum.at[dst], gsem))
            gathers.append(pltpu.async_copy(im_hbm.at[iidx_v.at[j]], bim.at[dst], gsem))
        for c in gathers:
            c.wait()
        out_sl = pl.ds(base, b_per_w)
        writes = [
            pltpu.async_copy(bug, oug.at[out_sl], ssem),
            pltpu.async_copy(big, oig.at[out_sl], ssem),
            pltpu.async_copy(bum, oum.at[out_sl], ssem),
            pltpu.async_copy(bim, oim.at[out_sl], ssem),
        ]
        for c in writes:
            c.wait()

    return gk


def _dense_body(ug_ref, ig_ref, um_ref, im_ref,
                w0, b0, w1, b1, w2, b2, w3, b3, wpg, wph, bp, out_ref):
    gmf = ug_ref[...] * ig_ref[...]
    h = jnp.concatenate([um_ref[...], im_ref[...]], axis=1)
    for w, b in ((w0, b0), (w1, b1), (w2, b2), (w3, b3)):
        h = jnp.maximum(
            jnp.dot(h, w[...], preferred_element_type=jnp.float32) + b[...], 0.0)
    pred = (jnp.dot(gmf, wpg[...], preferred_element_type=jnp.float32)
            + jnp.dot(h, wph[...], preferred_element_type=jnp.float32)
            + bp[...])
    out_ref[...] = jax.nn.sigmoid(pred)


def _dense(ug, ig, um, im, w0t, b0, w1t, b1, w2t, b2, w3t, b3, wpg, wph, bp):
    blk = 2048
    grid = (_BATCH // blk,)
    row = lambda i: (i, 0)
    fix = lambda i: (0, 0)
    fix1 = lambda i: (0,)
    in_specs = [
        pl.BlockSpec((blk, _D), row),
        pl.BlockSpec((blk, _D), row),
        pl.BlockSpec((blk, _D), row),
        pl.BlockSpec((blk, _D), row),
        pl.BlockSpec(w0t.shape, fix), pl.BlockSpec(b0.shape, fix1),
        pl.BlockSpec(w1t.shape, fix), pl.BlockSpec(b1.shape, fix1),
        pl.BlockSpec(w2t.shape, fix), pl.BlockSpec(b2.shape, fix1),
        pl.BlockSpec(w3t.shape, fix), pl.BlockSpec(b3.shape, fix1),
        pl.BlockSpec(wpg.shape, fix),
        pl.BlockSpec(wph.shape, fix),
        pl.BlockSpec(bp.shape, fix1),
    ]
    return pl.pallas_call(
        _dense_body,
        grid=grid,
        in_specs=in_specs,
        out_specs=pl.BlockSpec((blk, 1), row),
        out_shape=jax.ShapeDtypeStruct((_BATCH, 1), jnp.float32),
    )(ug, ig, um, im, w0t, b0, w1t, b1, w2t, b2, w3t, b3, wpg, wph, bp)


def kernel(user_indices, item_indices, ue_gmf, ie_gmf, ue_mlp, ie_mlp,
           W0, b0, W1, b1, W2, b2, W3, b3, Wp, bp):
    ui = user_indices.astype(jnp.int32)
    ii = item_indices.astype(jnp.int32)
    ug, ig, um, im = _gather4()(ui, ii, ue_gmf, ie_gmf, ue_mlp, ie_mlp)
    wpg = Wp[0, :_D].reshape(_D, 1)
    wph = Wp[0, _D:].reshape(-1, 1)
    return _dense(ug, ig, um, im,
                  W0.T, b0, W1.T, b1, W2.T, b2, W3.T, b3, wpg, wph, bp)
